# baseline (speedup 1.0000x reference)
import dataclasses
import functools

import jax
import jax.numpy as jnp
from jax import lax
from jax.experimental import pallas as pl
from jax.experimental.pallas import tpu as pltpu
from jax.experimental.pallas import tpu_sc as plsc

EXP = 8
TOPK = 2
DM = 1024
HD = 2048
SEQ = 2048
BM = 256
NBLK = SEQ * TOPK // BM + EXP
PTOT = NBLK * BM

NC, NS, LANES = 2, 16, 16
NW = NC * NS
CHUNK = SEQ // NW
HALF = 16
NWIN = CHUNK // HALF
RING = 3


def _routing_body(x_ref, wg_ref, bg_ref, dest_ref, w01_ref, meta_ref):
    x = x_ref[0]
    wg = wg_ref[...]
    lt = lax.dot_general(wg, x, dimension_numbers=(((0,), (1,)), ((), ())),
                         preferred_element_type=jnp.float32)
    lt = lt + bg_ref[...]

    eio = lax.broadcasted_iota(jnp.int32, (EXP, SEQ), 0)
    m1 = jnp.max(lt, axis=0, keepdims=True)
    a1 = jnp.min(jnp.where(lt == m1, eio, EXP), axis=0, keepdims=True)
    lt2 = jnp.where(eio == a1, -jnp.inf, lt)
    m2 = jnp.max(lt2, axis=0, keepdims=True)
    a2 = jnp.min(jnp.where(lt2 == m2, eio, EXP), axis=0, keepdims=True)
    e21 = jnp.exp(m2 - m1)
    w1 = 1.0 / (1.0 + e21)
    w2 = e21 / (1.0 + e21)

    keys = jnp.concatenate([a1, a2], axis=0)
    tmat = (lax.broadcasted_iota(jnp.int32, (SEQ, SEQ), 0)
            <= lax.broadcasted_iota(jnp.int32, (SEQ, SEQ), 1)).astype(jnp.float32)
    rowsel = lax.broadcasted_iota(jnp.int32, (TOPK, SEQ), 0)
    js = (lax.broadcasted_iota(jnp.int32, (1, 128), 1) * BM
          ).astype(jnp.float32)

    dest = jnp.zeros((TOPK, SEQ), jnp.float32)
    be = jnp.zeros((1, 128), jnp.float32)
    slotv = jnp.zeros((1, 128), jnp.float32)
    lastv = jnp.zeros((1, 1), jnp.float32)
    off = jnp.zeros((1, 1), jnp.float32)
    runcnt = jnp.zeros((1, 1), jnp.float32)
    masks = jnp.concatenate(
        [(keys == e).astype(jnp.float32) for e in range(EXP)], axis=0)
    p_all = lax.dot_general(masks, tmat,
                            dimension_numbers=(((1,), (0,)), ((), ())),
                            preferred_element_type=jnp.float32)
    inds, presents = [], []
    for e in range(EXP):
        m = masks[TOPK * e:TOPK * (e + 1)]
        p = p_all[TOPK * e:TOPK * (e + 1)]
        row0tot = p[0:1, SEQ - 1:SEQ]
        rank = p + jnp.where(rowsel == 1, 1.0, 0.0) * row0tot
        cnt = p[1:2, SEQ - 1:SEQ] + row0tot
        pc = jnp.ceil(cnt / BM) * BM
        present = (pc > 0).astype(jnp.float32)
        dest = jnp.where(m > 0, off + rank - 1.0, dest)
        ind = jnp.where((js >= off) & (js < off + pc), 1.0, 0.0)
        be = be + ind * float(e)
        slot_e = runcnt - 2.0 * jnp.floor(runcnt * 0.5)
        slotv = slotv + ind * slot_e
        lastv = jnp.where(pc > 0, jnp.full((1, 1), float(e)), lastv)
        off = off + pc
        runcnt = runcnt + present
        inds.append(ind)
        presents.append(present)
    nxtv = jnp.zeros((1, 128), jnp.float32)
    nxt = jnp.full((1, 1), -1.0)
    for e in range(EXP - 1, -1, -1):
        nxtv = nxtv + inds[e] * nxt
        nxt = jnp.where(presents[e] > 0, jnp.full((1, 1), float(e)), nxt)
    valid = (js < off).astype(jnp.int32)
    be = jnp.where(js < off, be, lastv)
    nxtv = jnp.where(js < off, nxtv, -1.0)
    meta_ref[...] = jnp.concatenate(
        [be.astype(jnp.int32), valid, slotv.astype(jnp.int32),
         nxtv.astype(jnp.int32)], axis=0)
    dest_ref[...] = dest.astype(jnp.int32)
    w01_ref[...] = jnp.concatenate([w1, w2], axis=0)


_routing = pl.pallas_call(
    _routing_body,
    out_shape=(
        jax.ShapeDtypeStruct((TOPK, SEQ), jnp.int32),
        jax.ShapeDtypeStruct((TOPK, SEQ), jnp.float32),
        jax.ShapeDtypeStruct((4, 128), jnp.int32),
    ),
)


def _dispatch_body(x_hbm, dest_hbm, xs_hbm, rows_v, d0_v, d1_v, sems):
    wid = lax.axis_index("s") * NC + lax.axis_index("c")
    base = wid * CHUNK
    pltpu.make_async_copy(dest_hbm.at[0, pl.ds(base, CHUNK)], d0_v,
                          sems.at[0]).start()
    pltpu.make_async_copy(dest_hbm.at[1, pl.ds(base, CHUNK)], d1_v,
                          sems.at[1]).start()
    pltpu.sync_copy(x_hbm.at[0, pl.ds(base, CHUNK)], rows_v)
    pltpu.make_async_copy(dest_hbm.at[0, pl.ds(base, CHUNK)], d0_v,
                          sems.at[0]).wait()
    pltpu.make_async_copy(dest_hbm.at[1, pl.ds(base, CHUNK)], d1_v,
                          sems.at[1]).wait()
    pltpu.make_async_copy(rows_v, xs_hbm.at[d0_v], sems.at[0]).start()
    pltpu.make_async_copy(rows_v, xs_hbm.at[d1_v], sems.at[1]).start()
    pltpu.make_async_copy(rows_v, xs_hbm.at[d0_v], sems.at[0]).wait()
    pltpu.make_async_copy(rows_v, xs_hbm.at[d1_v], sems.at[1]).wait()


@functools.lru_cache(maxsize=None)
def _sc_kernels():
    mesh = plsc.VectorSubcoreMesh(core_axis_name="c", subcore_axis_name="s")
    dispatch = pl.kernel(
        _dispatch_body,
        out_type=jax.ShapeDtypeStruct((PTOT, DM), jnp.float32),
        mesh=mesh,
        scratch_types=[
            pltpu.VMEM((CHUNK, DM), jnp.float32),
            pltpu.VMEM((CHUNK,), jnp.int32),
            pltpu.VMEM((CHUNK,), jnp.int32),
            pltpu.SemaphoreType.DMA((2,)),
        ],
    )
    cp = pltpu.CompilerParams()
    if "needs_layout_passes" in pltpu.CompilerParams.__dataclass_fields__:
        cp = dataclasses.replace(cp, needs_layout_passes=False)
    combine = pl.kernel(
        _combine_body,
        out_type=jax.ShapeDtypeStruct((SEQ, DM), jnp.float32),
        mesh=mesh,
        compiler_params=cp,
        scratch_types=[
            pltpu.VMEM((CHUNK,), jnp.int32),
            pltpu.VMEM((CHUNK,), jnp.int32),
            pltpu.VMEM((CHUNK,), jnp.float32),
            pltpu.VMEM((CHUNK,), jnp.float32),
            pltpu.VMEM((RING, HALF, DM), jnp.float32),
            pltpu.VMEM((RING, HALF, DM), jnp.float32),
            pltpu.SemaphoreType.DMA((RING, 2)),
            pltpu.SemaphoreType.DMA((RING,)),
        ],
    )
    return dispatch, combine


def _ffn_body(meta_ref, xs_ref, w0_hbm, b0_ref, w1_hbm, b1_ref, w2_hbm, b2_ref,
              y_ref, w0a, w1a, w2a, w0b, w1b, w2b, sems):
    j = pl.program_id(0)
    e = meta_ref[0, j]
    slot = meta_ref[2, j]
    nxt = meta_ref[3, j]
    boundary = (j == 0) | (e != meta_ref[0, jnp.maximum(j - 1, 0)])
    slots = ((w0a, w1a, w2a), (w0b, w1b, w2b))

    def issue(s, expert):
        r0, r1, r2 = slots[s]
        pltpu.make_async_copy(w0_hbm.at[expert], r0, sems.at[s, 0]).start()
        pltpu.make_async_copy(w1_hbm.at[expert], r1, sems.at[s, 1]).start()
        pltpu.make_async_copy(w2_hbm.at[expert], r2, sems.at[s, 2]).start()

    def drain(s, expert):
        r0, r1, r2 = slots[s]
        pltpu.make_async_copy(w0_hbm.at[expert], r0, sems.at[s, 0]).wait()
        pltpu.make_async_copy(w1_hbm.at[expert], r1, sems.at[s, 1]).wait()
        pltpu.make_async_copy(w2_hbm.at[expert], r2, sems.at[s, 2]).wait()

    @pl.when(j == 0)
    def _():
        issue(0, e)

    @pl.when(boundary & (nxt >= 0) & (slot == 0))
    def _():
        issue(1, nxt)

    @pl.when(boundary & (nxt >= 0) & (slot == 1))
    def _():
        issue(0, nxt)

    @pl.when(boundary & (slot == 0))
    def _():
        drain(0, e)

    @pl.when(boundary & (slot == 1))
    def _():
        drain(1, e)

    def compute(w0r, w1r, w2r):
        xb = xs_ref[...]
        h0 = jnp.dot(xb, w0r[...], preferred_element_type=jnp.float32,
                     precision=lax.Precision.DEFAULT)
        h0 = h0 + b0_ref[0]
        h1 = jnp.dot(xb, w1r[...], preferred_element_type=jnp.float32,
                     precision=lax.Precision.DEFAULT)
        h1 = h1 + b1_ref[0]
        g = h0 * (h1 * jax.nn.sigmoid(h1))
        y = jnp.dot(g, w2r[...], preferred_element_type=jnp.float32,
                    precision=lax.Precision.DEFAULT)
        y_ref[...] = y + b2_ref[0]

    valid = meta_ref[1, j] == 1

    @pl.when(valid & (slot == 0))
    def _():
        compute(w0a, w1a, w2a)

    @pl.when(valid & (slot == 1))
    def _():
        compute(w0b, w1b, w2b)


_ffn = pl.pallas_call(
    _ffn_body,
    grid_spec=pltpu.PrefetchScalarGridSpec(
        num_scalar_prefetch=1,
        grid=(NBLK,),
        in_specs=[
            pl.BlockSpec((BM, DM), lambda j, meta: (j, 0)),
            pl.BlockSpec(memory_space=pl.ANY),
            pl.BlockSpec((1, 1, HD), lambda j, meta: (meta[0, j], 0, 0)),
            pl.BlockSpec(memory_space=pl.ANY),
            pl.BlockSpec((1, 1, HD), lambda j, meta: (meta[0, j], 0, 0)),
            pl.BlockSpec(memory_space=pl.ANY),
            pl.BlockSpec((1, 1, DM), lambda j, meta: (meta[0, j], 0, 0)),
        ],
        out_specs=pl.BlockSpec((BM, DM), lambda j, meta: (j, 0)),
        scratch_shapes=[
            pltpu.VMEM((DM, HD), jnp.float32),
            pltpu.VMEM((DM, HD), jnp.float32),
            pltpu.VMEM((HD, DM), jnp.float32),
            pltpu.VMEM((DM, HD), jnp.float32),
            pltpu.VMEM((DM, HD), jnp.float32),
            pltpu.VMEM((HD, DM), jnp.float32),
            pltpu.SemaphoreType.DMA((2, 3)),
        ],
    ),
    out_shape=jax.ShapeDtypeStruct((PTOT, DM), jnp.float32),
    compiler_params=pltpu.CompilerParams(
        dimension_semantics=("arbitrary",)),
)


def _combine_body(y_hbm, dest_hbm, w01_hbm, out_hbm,
                  d0_v, d1_v, w0_v, w1_v, r0, r1, gsem, ssem):
    wid = lax.axis_index("s") * NC + lax.axis_index("c")
    base = wid * CHUNK
    pltpu.sync_copy(dest_hbm.at[0, pl.ds(base, CHUNK)], d0_v)
    pltpu.sync_copy(dest_hbm.at[1, pl.ds(base, CHUNK)], d1_v)
    pltpu.sync_copy(w01_hbm.at[0, pl.ds(base, CHUNK)], w0_v)
    pltpu.sync_copy(w01_hbm.at[1, pl.ds(base, CHUNK)], w1_v)

    def start_gathers(h, p):
        i0 = d0_v.at[pl.ds(h * HALF, HALF)]
        i1 = d1_v.at[pl.ds(h * HALF, HALF)]
        pltpu.make_async_copy(y_hbm.at[i0], r0.at[p], gsem.at[p, 0]).start()
        pltpu.make_async_copy(y_hbm.at[i1], r1.at[p], gsem.at[p, 1]).start()

    def wait_gathers(h, p):
        i0 = d0_v.at[pl.ds(h * HALF, HALF)]
        i1 = d1_v.at[pl.ds(h * HALF, HALF)]
        pltpu.make_async_copy(y_hbm.at[i0], r0.at[p], gsem.at[p, 0]).wait()
        pltpu.make_async_copy(y_hbm.at[i1], r1.at[p], gsem.at[p, 1]).wait()

    def store_desc(h, p):
        dst = out_hbm.at[pl.ds(base + h * HALF, HALF)]
        return pltpu.make_async_copy(r0.at[p], dst, ssem.at[p])

    start_gathers(0, 0)
    start_gathers(1, 1)
    stores_waited = -1
    for h in range(NWIN):
        p = h % RING
        wait_gathers(h, p)
        if h + 2 < NWIN:
            if h >= 1:
                store_desc(h - 1, (h - 1) % RING).wait()
                stores_waited = h - 1
            start_gathers(h + 2, (h + 2) % RING)

        @pl.loop(0, HALF)
        def _(r):
            ridx = jnp.full((LANES,), h * HALF, jnp.int32) + r
            wa = plsc.load_gather(w0_v, [ridx])
            wb = plsc.load_gather(w1_v, [ridx])

            @pl.loop(0, DM, step=LANES)
            def _(c):
                va = r0[p, r, pl.ds(c, LANES)]
                vb = r1[p, r, pl.ds(c, LANES)]
                r0[p, r, pl.ds(c, LANES)] = wa * va + wb * vb

        store_desc(h, p).start()
    for h in range(stores_waited + 1, NWIN):
        store_desc(h, h % RING).wait()


def kernel(x, Wg, bg, W0, b0, W1, b1, W2, b2):
    dispatch, combine = _sc_kernels()
    dest, w01, meta = _routing(x, Wg, bg.reshape(EXP, 1))
    xs = dispatch(x, dest)
    y = _ffn(meta, xs,
             W0, b0.reshape(EXP, 1, HD),
             W1, b1.reshape(EXP, 1, HD),
             W2, b2.reshape(EXP, 1, DM))
    out = combine(y, dest, w01)
    return out.reshape(1, SEQ, DM)


# speedup vs baseline: 1.1113x; 1.1113x over previous
import dataclasses
import functools

import jax
import jax.numpy as jnp
from jax import lax
from jax.experimental import pallas as pl
from jax.experimental.pallas import tpu as pltpu
from jax.experimental.pallas import tpu_sc as plsc

EXP = 8
TOPK = 2
DM = 1024
HD = 2048
SEQ = 2048
BM = 256
NBLK = SEQ * TOPK // BM + EXP
PTOT = NBLK * BM

NC, NS, LANES = 2, 16, 16
NW = NC * NS
CHUNK = SEQ // NW
HALF = 16
NWIN = CHUNK // HALF
RING = 3


def _routing_body(x_ref, wg_ref, bg_ref, dest_ref, w01_ref, meta_ref):
    x = x_ref[0]
    wg = wg_ref[...]
    lt = lax.dot_general(wg, x, dimension_numbers=(((0,), (1,)), ((), ())),
                         preferred_element_type=jnp.float32)
    lt = lt + bg_ref[...]

    eio = lax.broadcasted_iota(jnp.int32, (EXP, SEQ), 0)
    m1 = jnp.max(lt, axis=0, keepdims=True)
    a1 = jnp.min(jnp.where(lt == m1, eio, EXP), axis=0, keepdims=True)
    lt2 = jnp.where(eio == a1, -jnp.inf, lt)
    m2 = jnp.max(lt2, axis=0, keepdims=True)
    a2 = jnp.min(jnp.where(lt2 == m2, eio, EXP), axis=0, keepdims=True)
    e21 = jnp.exp(m2 - m1)
    w1 = 1.0 / (1.0 + e21)
    w2 = e21 / (1.0 + e21)

    keys = jnp.concatenate([a1, a2], axis=0)
    tmat = (lax.broadcasted_iota(jnp.int32, (SEQ, SEQ), 0)
            <= lax.broadcasted_iota(jnp.int32, (SEQ, SEQ), 1)).astype(jnp.float32)
    rowsel = lax.broadcasted_iota(jnp.int32, (TOPK, SEQ), 0)
    js = (lax.broadcasted_iota(jnp.int32, (1, 128), 1) * BM
          ).astype(jnp.float32)

    dest = jnp.zeros((TOPK, SEQ), jnp.float32)
    be = jnp.zeros((1, 128), jnp.float32)
    slotv = jnp.zeros((1, 128), jnp.float32)
    lastv = jnp.zeros((1, 1), jnp.float32)
    off = jnp.zeros((1, 1), jnp.float32)
    runcnt = jnp.zeros((1, 1), jnp.float32)
    masks = jnp.concatenate(
        [(keys == e).astype(jnp.float32) for e in range(EXP)], axis=0)
    p_all = lax.dot_general(masks, tmat,
                            dimension_numbers=(((1,), (0,)), ((), ())),
                            preferred_element_type=jnp.float32)
    inds, presents = [], []
    for e in range(EXP):
        m = masks[TOPK * e:TOPK * (e + 1)]
        p = p_all[TOPK * e:TOPK * (e + 1)]
        row0tot = p[0:1, SEQ - 1:SEQ]
        rank = p + jnp.where(rowsel == 1, 1.0, 0.0) * row0tot
        cnt = p[1:2, SEQ - 1:SEQ] + row0tot
        pc = jnp.ceil(cnt / BM) * BM
        present = (pc > 0).astype(jnp.float32)
        dest = jnp.where(m > 0, off + rank - 1.0, dest)
        ind = jnp.where((js >= off) & (js < off + pc), 1.0, 0.0)
        be = be + ind * float(e)
        slot_e = runcnt - 2.0 * jnp.floor(runcnt * 0.5)
        slotv = slotv + ind * slot_e
        lastv = jnp.where(pc > 0, jnp.full((1, 1), float(e)), lastv)
        off = off + pc
        runcnt = runcnt + present
        inds.append(ind)
        presents.append(present)
    nxtv = jnp.zeros((1, 128), jnp.float32)
    nxt = jnp.full((1, 1), -1.0)
    for e in range(EXP - 1, -1, -1):
        nxtv = nxtv + inds[e] * nxt
        nxt = jnp.where(presents[e] > 0, jnp.full((1, 1), float(e)), nxt)
    valid = (js < off).astype(jnp.int32)
    be = jnp.where(js < off, be, lastv)
    nxtv = jnp.where(js < off, nxtv, -1.0)
    meta_ref[...] = jnp.concatenate(
        [be.astype(jnp.int32), valid, slotv.astype(jnp.int32),
         nxtv.astype(jnp.int32)], axis=0)
    dest_ref[...] = dest.astype(jnp.int32)
    w01_ref[...] = jnp.concatenate([w1, w2], axis=0)


_routing = pl.pallas_call(
    _routing_body,
    out_shape=(
        jax.ShapeDtypeStruct((TOPK, SEQ), jnp.int32),
        jax.ShapeDtypeStruct((TOPK, SEQ), jnp.float32),
        jax.ShapeDtypeStruct((4, 128), jnp.int32),
    ),
)


def _dispatch_body(x_hbm, dest_hbm, xs_hbm, rows_v, d0_v, d1_v, sems):
    wid = lax.axis_index("s") * NC + lax.axis_index("c")
    base = wid * CHUNK
    pltpu.make_async_copy(dest_hbm.at[0, pl.ds(base, CHUNK)], d0_v,
                          sems.at[0]).start()
    pltpu.make_async_copy(dest_hbm.at[1, pl.ds(base, CHUNK)], d1_v,
                          sems.at[1]).start()
    pltpu.sync_copy(x_hbm.at[0, pl.ds(base, CHUNK)], rows_v)
    pltpu.make_async_copy(dest_hbm.at[0, pl.ds(base, CHUNK)], d0_v,
                          sems.at[0]).wait()
    pltpu.make_async_copy(dest_hbm.at[1, pl.ds(base, CHUNK)], d1_v,
                          sems.at[1]).wait()
    pltpu.make_async_copy(rows_v, xs_hbm.at[d0_v], sems.at[0]).start()
    pltpu.make_async_copy(rows_v, xs_hbm.at[d1_v], sems.at[1]).start()
    pltpu.make_async_copy(rows_v, xs_hbm.at[d0_v], sems.at[0]).wait()
    pltpu.make_async_copy(rows_v, xs_hbm.at[d1_v], sems.at[1]).wait()


@functools.lru_cache(maxsize=None)
def _sc_kernels():
    mesh = plsc.VectorSubcoreMesh(core_axis_name="c", subcore_axis_name="s")
    dispatch = pl.kernel(
        _dispatch_body,
        out_type=jax.ShapeDtypeStruct((PTOT, DM), jnp.float32),
        mesh=mesh,
        scratch_types=[
            pltpu.VMEM((CHUNK, DM), jnp.float32),
            pltpu.VMEM((CHUNK,), jnp.int32),
            pltpu.VMEM((CHUNK,), jnp.int32),
            pltpu.SemaphoreType.DMA((2,)),
        ],
    )
    cp = pltpu.CompilerParams()
    if "needs_layout_passes" in pltpu.CompilerParams.__dataclass_fields__:
        cp = dataclasses.replace(cp, needs_layout_passes=False)
    combine = pl.kernel(
        _combine_body,
        out_type=jax.ShapeDtypeStruct((SEQ, DM), jnp.float32),
        mesh=mesh,
        compiler_params=cp,
        scratch_types=[
            pltpu.VMEM((CHUNK,), jnp.int32),
            pltpu.VMEM((CHUNK,), jnp.int32),
            pltpu.VMEM((CHUNK,), jnp.float32),
            pltpu.VMEM((CHUNK,), jnp.float32),
            pltpu.VMEM((RING, HALF, DM), jnp.float32),
            pltpu.VMEM((RING, HALF, DM), jnp.float32),
            pltpu.SemaphoreType.DMA((RING, 2)),
            pltpu.SemaphoreType.DMA((RING,)),
        ],
    )
    return dispatch, combine


def _ffn_body(meta_ref, xs_ref, w0_hbm, b0_ref, w1_hbm, b1_ref, w2_hbm, b2_ref,
              y_ref, w0a, w1a, w2a, w0b, w1b, w2b, sems):
    j = pl.program_id(0)
    e = meta_ref[0, j]
    slot = meta_ref[2, j]
    nxt = meta_ref[3, j]
    boundary = (j == 0) | (e != meta_ref[0, jnp.maximum(j - 1, 0)])
    slots = ((w0a, w1a, w2a), (w0b, w1b, w2b))

    def issue(s, expert):
        r0, r1, r2 = slots[s]
        pltpu.make_async_copy(w0_hbm.at[expert], r0, sems.at[s, 0]).start()
        pltpu.make_async_copy(w1_hbm.at[expert], r1, sems.at[s, 1]).start()
        pltpu.make_async_copy(w2_hbm.at[expert], r2, sems.at[s, 2]).start()

    def drain(s, expert):
        r0, r1, r2 = slots[s]
        pltpu.make_async_copy(w0_hbm.at[expert], r0, sems.at[s, 0]).wait()
        pltpu.make_async_copy(w1_hbm.at[expert], r1, sems.at[s, 1]).wait()
        pltpu.make_async_copy(w2_hbm.at[expert], r2, sems.at[s, 2]).wait()

    @pl.when(j == 0)
    def _():
        issue(0, e)

    @pl.when(boundary & (nxt >= 0) & (slot == 0))
    def _():
        issue(1, nxt)

    @pl.when(boundary & (nxt >= 0) & (slot == 1))
    def _():
        issue(0, nxt)

    @pl.when(boundary & (slot == 0))
    def _():
        drain(0, e)

    @pl.when(boundary & (slot == 1))
    def _():
        drain(1, e)

    def compute(w0r, w1r, w2r):
        xb = xs_ref[...]
        h0 = jnp.dot(xb, w0r[...], preferred_element_type=jnp.float32,
                     precision=lax.Precision.DEFAULT)
        h0 = h0 + b0_ref[0]
        h1 = jnp.dot(xb, w1r[...], preferred_element_type=jnp.float32,
                     precision=lax.Precision.DEFAULT)
        h1 = h1 + b1_ref[0]
        g = h0 * (h1 * jax.nn.sigmoid(h1))
        y = jnp.dot(g, w2r[...], preferred_element_type=jnp.float32,
                    precision=lax.Precision.DEFAULT)
        y_ref[...] = y + b2_ref[0]

    valid = meta_ref[1, j] == 1

    @pl.when(valid & (slot == 0))
    def _():
        compute(w0a, w1a, w2a)

    @pl.when(valid & (slot == 1))
    def _():
        compute(w0b, w1b, w2b)


_ffn = pl.pallas_call(
    _ffn_body,
    grid_spec=pltpu.PrefetchScalarGridSpec(
        num_scalar_prefetch=1,
        grid=(NBLK,),
        in_specs=[
            pl.BlockSpec((BM, DM), lambda j, meta: (j, 0)),
            pl.BlockSpec(memory_space=pl.ANY),
            pl.BlockSpec((1, 1, HD), lambda j, meta: (meta[0, j], 0, 0)),
            pl.BlockSpec(memory_space=pl.ANY),
            pl.BlockSpec((1, 1, HD), lambda j, meta: (meta[0, j], 0, 0)),
            pl.BlockSpec(memory_space=pl.ANY),
            pl.BlockSpec((1, 1, DM), lambda j, meta: (meta[0, j], 0, 0)),
        ],
        out_specs=pl.BlockSpec((BM, DM), lambda j, meta: (j, 0)),
        scratch_shapes=[
            pltpu.VMEM((DM, HD), jnp.float32),
            pltpu.VMEM((DM, HD), jnp.float32),
            pltpu.VMEM((HD, DM), jnp.float32),
            pltpu.VMEM((DM, HD), jnp.float32),
            pltpu.VMEM((DM, HD), jnp.float32),
            pltpu.VMEM((HD, DM), jnp.float32),
            pltpu.SemaphoreType.DMA((2, 3)),
        ],
    ),
    out_shape=jax.ShapeDtypeStruct((PTOT, DM), jnp.float32),
    compiler_params=pltpu.CompilerParams(
        dimension_semantics=("arbitrary",)),
)


def _combine_body(y_hbm, dest_hbm, w01_hbm, out_hbm,
                  d0_v, d1_v, w0_v, w1_v, r0, r1, gsem, ssem):
    wid = lax.axis_index("s") * NC + lax.axis_index("c")
    base = wid * CHUNK
    pltpu.sync_copy(dest_hbm.at[0, pl.ds(base, CHUNK)], d0_v)
    pltpu.sync_copy(dest_hbm.at[1, pl.ds(base, CHUNK)], d1_v)
    pltpu.sync_copy(w01_hbm.at[0, pl.ds(base, CHUNK)], w0_v)
    pltpu.sync_copy(w01_hbm.at[1, pl.ds(base, CHUNK)], w1_v)

    def start_gathers(h, p):
        i0 = d0_v.at[pl.ds(h * HALF, HALF)]
        i1 = d1_v.at[pl.ds(h * HALF, HALF)]
        pltpu.make_async_copy(y_hbm.at[i0], r0.at[p], gsem.at[p, 0]).start()
        pltpu.make_async_copy(y_hbm.at[i1], r1.at[p], gsem.at[p, 1]).start()

    def wait_gathers(h, p):
        i0 = d0_v.at[pl.ds(h * HALF, HALF)]
        i1 = d1_v.at[pl.ds(h * HALF, HALF)]
        pltpu.make_async_copy(y_hbm.at[i0], r0.at[p], gsem.at[p, 0]).wait()
        pltpu.make_async_copy(y_hbm.at[i1], r1.at[p], gsem.at[p, 1]).wait()

    def store_desc(h, p):
        dst = out_hbm.at[pl.ds(base + h * HALF, HALF)]
        return pltpu.make_async_copy(r0.at[p], dst, ssem.at[p])

    start_gathers(0, 0)
    start_gathers(1, 1)
    stores_waited = -1
    for h in range(NWIN):
        p = h % RING
        wait_gathers(h, p)
        if h + 2 < NWIN:
            if h >= 1:
                store_desc(h - 1, (h - 1) % RING).wait()
                stores_waited = h - 1
            start_gathers(h + 2, (h + 2) % RING)

        @pl.loop(0, HALF)
        def _(r):
            ridx = jnp.full((LANES,), h * HALF, jnp.int32) + r
            wa = plsc.load_gather(w0_v, [ridx])
            wb = plsc.load_gather(w1_v, [ridx])

            @pl.loop(0, DM, step=4 * LANES)
            def _(c):
                for k in range(4):
                    ck = c + k * LANES
                    va = r0[p, r, pl.ds(ck, LANES)]
                    vb = r1[p, r, pl.ds(ck, LANES)]
                    r0[p, r, pl.ds(ck, LANES)] = wa * va + wb * vb

        store_desc(h, p).start()
    for h in range(stores_waited + 1, NWIN):
        store_desc(h, h % RING).wait()


def kernel(x, Wg, bg, W0, b0, W1, b1, W2, b2):
    dispatch, combine = _sc_kernels()
    dest, w01, meta = _routing(x, Wg, bg.reshape(EXP, 1))
    xs = dispatch(x, dest)
    y = _ffn(meta, xs,
             W0, b0.reshape(EXP, 1, HD),
             W1, b1.reshape(EXP, 1, HD),
             W2, b2.reshape(EXP, 1, DM))
    out = combine(y, dest, w01)
    return out.reshape(1, SEQ, DM)
